# baseline (speedup 1.0000x reference)
import jax
import jax.numpy as jnp
from jax import lax
from jax.experimental import pallas as pl
from jax.experimental.pallas import tpu as pltpu
from jax.experimental.pallas import tpu_sc as plsc

NUM_FEATURES = 26
VOCAB = 100000
EMBED_DIM = 32
BATCH = 16384

NUM_CORES = 2
NUM_SUBCORES = 16
NUM_WORKERS = NUM_CORES * NUM_SUBCORES

CHUNK = 8192
NCH = BATCH // CHUNK


def _sc_body(idx_hbm, tab_hbm, out_hbm, row_v, idx_v, out_v, sem):
    wid = lax.axis_index("s") * NUM_CORES + lax.axis_index("c")
    r = wid

    for f in range(NUM_FEATURES):
        pltpu.sync_copy(tab_hbm.at[f, r], row_v)
        for c in range(NCH):
            pltpu.sync_copy(idx_hbm.at[f, pl.ds(c * CHUNK, CHUNK)], idx_v)

            def jbody(j, carry):
                iv = idx_v[pl.ds(j * 16, 16)]
                out_v[pl.ds(j * 16, 16)] = plsc.load_gather(row_v, [iv])
                return carry

            lax.fori_loop(0, CHUNK // 16, jbody, 0, unroll=8)
            pltpu.sync_copy(out_v, out_hbm.at[f, r, pl.ds(c * CHUNK, CHUNK)])


@jax.jit
def kernel(inputs, tables):
    tables_t = tables.transpose(0, 2, 1)
    inputs_t = inputs.T.astype(jnp.int32)
    run = pl.kernel(
        _sc_body,
        out_type=jax.ShapeDtypeStruct((NUM_FEATURES, EMBED_DIM, BATCH), jnp.float32),
        mesh=plsc.VectorSubcoreMesh(core_axis_name="c", subcore_axis_name="s"),
        compiler_params=pltpu.CompilerParams(needs_layout_passes=False),
        scratch_types=[
            pltpu.VMEM((VOCAB,), jnp.float32),
            pltpu.VMEM((CHUNK,), jnp.int32),
            pltpu.VMEM((CHUNK,), jnp.float32),
            pltpu.SemaphoreType.DMA,
        ],
    )
    out_t = run(inputs_t, tables_t)
    return out_t.transpose(0, 2, 1)


# speedup vs baseline: 1.0925x; 1.0925x over previous
import jax
import jax.numpy as jnp
from jax import lax
from jax.experimental import pallas as pl
from jax.experimental.pallas import tpu as pltpu
from jax.experimental.pallas import tpu_sc as plsc

NUM_FEATURES = 26
VOCAB = 100000
EMBED_DIM = 32
BATCH = 16384

NUM_CORES = 2
NUM_SUBCORES = 16
NUM_WORKERS = NUM_CORES * NUM_SUBCORES

CHUNK = 4096
NCH = BATCH // CHUNK


def _sc_body(idx_hbm, tab_hbm, out_hbm, row_v, idx_v, ob0, ob1,
             sem_row, sem_idx, sem_wb0, sem_wb1):
    wid = lax.axis_index("s") * NUM_CORES + lax.axis_index("c")
    r = wid

    pltpu.async_copy(tab_hbm.at[0, r], row_v, sem_row)
    pltpu.async_copy(idx_hbm.at[0], idx_v, sem_idx)

    def gather_c(c, obuf):
        def jbody(j, carry):
            iv = idx_v[pl.ds(c * CHUNK + j * 16, 16)]
            obuf[pl.ds(j * 16, 16)] = plsc.load_gather(row_v, [iv])
            return carry

        lax.fori_loop(0, CHUNK // 16, jbody, 0)

    for f in range(NUM_FEATURES):
        pltpu.make_async_copy(tab_hbm.at[f, r], row_v, sem_row).wait()
        pltpu.make_async_copy(idx_hbm.at[f], idx_v, sem_idx).wait()

        gather_c(0, ob0)
        wb0a = pltpu.async_copy(ob0, out_hbm.at[f, r, pl.ds(0 * CHUNK, CHUNK)], sem_wb0)
        gather_c(1, ob1)
        wb1a = pltpu.async_copy(ob1, out_hbm.at[f, r, pl.ds(1 * CHUNK, CHUNK)], sem_wb1)
        wb0a.wait()
        gather_c(2, ob0)
        wb0b = pltpu.async_copy(ob0, out_hbm.at[f, r, pl.ds(2 * CHUNK, CHUNK)], sem_wb0)
        wb1a.wait()
        gather_c(3, ob1)
        wb1b = pltpu.async_copy(ob1, out_hbm.at[f, r, pl.ds(3 * CHUNK, CHUNK)], sem_wb1)

        if f < NUM_FEATURES - 1:
            pltpu.async_copy(tab_hbm.at[f + 1, r], row_v, sem_row)
            pltpu.async_copy(idx_hbm.at[f + 1], idx_v, sem_idx)

        wb0b.wait()
        wb1b.wait()


@jax.jit
def kernel(inputs, tables):
    tables_t = tables.transpose(0, 2, 1)
    inputs_t = inputs.T.astype(jnp.int32)
    run = pl.kernel(
        _sc_body,
        out_type=jax.ShapeDtypeStruct((NUM_FEATURES, EMBED_DIM, BATCH), jnp.float32),
        mesh=plsc.VectorSubcoreMesh(core_axis_name="c", subcore_axis_name="s"),
        compiler_params=pltpu.CompilerParams(needs_layout_passes=False),
        scratch_types=[
            pltpu.VMEM((VOCAB,), jnp.float32),
            pltpu.VMEM((BATCH,), jnp.int32),
            pltpu.VMEM((CHUNK,), jnp.float32),
            pltpu.VMEM((CHUNK,), jnp.float32),
            pltpu.SemaphoreType.DMA,
            pltpu.SemaphoreType.DMA,
            pltpu.SemaphoreType.DMA,
            pltpu.SemaphoreType.DMA,
        ],
    )
    out_t = run(inputs_t, tables_t)
    return out_t.transpose(0, 2, 1)


# speedup vs baseline: 1.6956x; 1.5520x over previous
import jax
import jax.numpy as jnp
from jax import lax
from jax.experimental import pallas as pl
from jax.experimental.pallas import tpu as pltpu
from jax.experimental.pallas import tpu_sc as plsc

NUM_FEATURES = 26
VOCAB = 100000
EMBED_DIM = 32
BATCH = 16384

NUM_CORES = 2
NUM_SUBCORES = 16
NUM_WORKERS = NUM_CORES * NUM_SUBCORES

CHUNK = 8192
NCH = BATCH // CHUNK


def _sc_body(idx_hbm, tab_hbm, out_hbm, row_v, idx_v, out_v, sem):
    wid = lax.axis_index("s") * NUM_CORES + lax.axis_index("c")
    r = wid

    for f in range(NUM_FEATURES):
        pltpu.sync_copy(tab_hbm.at[f, r], row_v)
        for c in range(NCH):
            pltpu.sync_copy(idx_hbm.at[f, pl.ds(c * CHUNK, CHUNK)], idx_v)

            @plsc.parallel_loop(0, CHUNK, step=16, unroll=4)
            def _(i):
                iv = idx_v[pl.ds(i, 16)]
                out_v[pl.ds(i, 16)] = plsc.load_gather(row_v, [iv])

            pltpu.sync_copy(out_v, out_hbm.at[f, r, pl.ds(c * CHUNK, CHUNK)])


@jax.jit
def kernel(inputs, tables):
    tables_t = tables.transpose(0, 2, 1)
    inputs_t = inputs.T.astype(jnp.int32)
    run = pl.kernel(
        _sc_body,
        out_type=jax.ShapeDtypeStruct((NUM_FEATURES, EMBED_DIM, BATCH), jnp.float32),
        mesh=plsc.VectorSubcoreMesh(core_axis_name="c", subcore_axis_name="s"),
        compiler_params=pltpu.CompilerParams(needs_layout_passes=False),
        scratch_types=[
            pltpu.VMEM((VOCAB,), jnp.float32),
            pltpu.VMEM((CHUNK,), jnp.int32),
            pltpu.VMEM((CHUNK,), jnp.float32),
            pltpu.SemaphoreType.DMA,
        ],
    )
    out_t = run(inputs_t, tables_t)
    return out_t.transpose(0, 2, 1)


# speedup vs baseline: 1.7038x; 1.0048x over previous
import jax
import jax.numpy as jnp
from jax import lax
from jax.experimental import pallas as pl
from jax.experimental.pallas import tpu as pltpu
from jax.experimental.pallas import tpu_sc as plsc

NUM_FEATURES = 26
VOCAB = 100000
EMBED_DIM = 32
BATCH = 16384

NUM_CORES = 2
NUM_SUBCORES = 16
NUM_WORKERS = NUM_CORES * NUM_SUBCORES

CHUNK = 8192
NCH = BATCH // CHUNK


def _sc_body(idx_hbm, tab_hbm, out_hbm, row_v, idx_v, out_v, sem):
    wid = lax.axis_index("s") * NUM_CORES + lax.axis_index("c")
    r = wid

    for f in range(NUM_FEATURES):
        pltpu.sync_copy(tab_hbm.at[f, r], row_v)
        for c in range(NCH):
            pltpu.sync_copy(idx_hbm.at[f, pl.ds(c * CHUNK, CHUNK)], idx_v)

            @plsc.parallel_loop(0, CHUNK, step=16, unroll=8)
            def _(i):
                iv = idx_v[pl.ds(i, 16)]
                out_v[pl.ds(i, 16)] = plsc.load_gather(row_v, [iv])

            pltpu.sync_copy(out_v, out_hbm.at[f, r, pl.ds(c * CHUNK, CHUNK)])


@jax.jit
def kernel(inputs, tables):
    tables_t = tables.transpose(0, 2, 1)
    inputs_t = inputs.T.astype(jnp.int32)
    run = pl.kernel(
        _sc_body,
        out_type=jax.ShapeDtypeStruct((NUM_FEATURES, EMBED_DIM, BATCH), jnp.float32),
        mesh=plsc.VectorSubcoreMesh(core_axis_name="c", subcore_axis_name="s"),
        compiler_params=pltpu.CompilerParams(needs_layout_passes=False),
        scratch_types=[
            pltpu.VMEM((VOCAB,), jnp.float32),
            pltpu.VMEM((CHUNK,), jnp.int32),
            pltpu.VMEM((CHUNK,), jnp.float32),
            pltpu.SemaphoreType.DMA,
        ],
    )
    out_t = run(inputs_t, tables_t)
    return out_t.transpose(0, 2, 1)


# speedup vs baseline: 1.8405x; 1.0802x over previous
import jax
import jax.numpy as jnp
from jax import lax
from jax.experimental import pallas as pl
from jax.experimental.pallas import tpu as pltpu
from jax.experimental.pallas import tpu_sc as plsc

NUM_FEATURES = 26
VOCAB = 100000
EMBED_DIM = 32
BATCH = 16384

NUM_CORES = 2
NUM_SUBCORES = 16
NUM_WORKERS = NUM_CORES * NUM_SUBCORES

CHUNK = 8192
NCH = BATCH // CHUNK
QCH = 4096
NQPC = CHUNK // QCH


def _sc_body(idx_hbm, tab_hbm, out_hbm, row_v, ib0, ib1, ob0, ob1,
             sem_row, sem_i0, sem_i1, sem_w0, sem_w1):
    wid = lax.axis_index("s") * NUM_CORES + lax.axis_index("c")
    r = wid

    idx_bufs = (ib0, ib1)
    idx_sems = (sem_i0, sem_i1)
    out_bufs = (ob0, ob1)
    out_sems = (sem_w0, sem_w1)
    pending_wb = [None, None]

    pltpu.async_copy(tab_hbm.at[0, r], row_v, sem_row)
    for c in range(NCH):
        pltpu.async_copy(idx_hbm.at[0, pl.ds(c * CHUNK, CHUNK)], idx_bufs[c], idx_sems[c])

    for f in range(NUM_FEATURES):
        pltpu.make_async_copy(tab_hbm.at[f, r], row_v, sem_row).wait()
        q = 0
        for c in range(NCH):
            ib = idx_bufs[c]
            pltpu.make_async_copy(
                idx_hbm.at[f, pl.ds(c * CHUNK, CHUNK)], ib, idx_sems[c]
            ).wait()
            for qq in range(NQPC):
                ob = out_bufs[q % 2]
                if pending_wb[q % 2] is not None:
                    pending_wb[q % 2].wait()
                base = qq * QCH

                @plsc.parallel_loop(0, QCH, step=16, unroll=8)
                def _(i):
                    iv = ib[pl.ds(base + i, 16)]
                    ob[pl.ds(i, 16)] = plsc.load_gather(row_v, [iv])

                pending_wb[q % 2] = pltpu.async_copy(
                    ob, out_hbm.at[f, r, pl.ds(q * QCH, QCH)], out_sems[q % 2]
                )
                q += 1
            if f < NUM_FEATURES - 1:
                pltpu.async_copy(
                    idx_hbm.at[f + 1, pl.ds(c * CHUNK, CHUNK)], ib, idx_sems[c]
                )
        if f < NUM_FEATURES - 1:
            pltpu.async_copy(tab_hbm.at[f + 1, r], row_v, sem_row)

    pending_wb[0].wait()
    pending_wb[1].wait()


@jax.jit
def kernel(inputs, tables):
    tables_t = tables.transpose(0, 2, 1)
    inputs_t = inputs.T.astype(jnp.int32)
    run = pl.kernel(
        _sc_body,
        out_type=jax.ShapeDtypeStruct((NUM_FEATURES, EMBED_DIM, BATCH), jnp.float32),
        mesh=plsc.VectorSubcoreMesh(core_axis_name="c", subcore_axis_name="s"),
        compiler_params=pltpu.CompilerParams(needs_layout_passes=False),
        scratch_types=[
            pltpu.VMEM((VOCAB,), jnp.float32),
            pltpu.VMEM((CHUNK,), jnp.int32),
            pltpu.VMEM((CHUNK,), jnp.int32),
            pltpu.VMEM((QCH,), jnp.float32),
            pltpu.VMEM((QCH,), jnp.float32),
            pltpu.SemaphoreType.DMA,
            pltpu.SemaphoreType.DMA,
            pltpu.SemaphoreType.DMA,
            pltpu.SemaphoreType.DMA,
            pltpu.SemaphoreType.DMA,
        ],
    )
    out_t = run(inputs_t, tables_t)
    return out_t.transpose(0, 2, 1)
